# baseline (speedup 1.0000x reference)
import functools

import jax
import jax.numpy as jnp
from jax import lax
from jax.experimental import pallas as pl
from jax.experimental.pallas import tpu as pltpu
from jax.experimental.pallas import tpu_sc as plsc

N_NODES = 10000
D = 128
E = 320000
N_GRAPHS = 128
N_LAYERS = 7
BETA = 0.85

NC, NS, L = 2, 16, 16
NW = NC * NS
CHUNK = 128
NBUF = 1
CPT0 = 79
CPT1 = 79
S0 = 80
S1 = 80
E_PAD = NS * (CPT0 + CPT1) * CHUNK
AGG_ROWS = 10112
STRIPE = AGG_ROWS // NS
SINK = N_NODES

def _sc_body(h_hbm, src_hbm, dst_hbm, out0, out1, agg_sh, src_v, dst_v, rows_v,
             rows_w, rows_x, rows_y, *sems):
    core = lax.axis_index("c")
    sub = lax.axis_index("s")
    wid = core * NS + sub

    def _zrow(i, carry):
        for c in range(D // L):
            rows_v[i, pl.ds(c * L, L)] = jnp.zeros((L,), jnp.float32)
        return carry

    ZR = CHUNK // 2
    lax.fori_loop(0, ZR, _zrow, 0)
    base = sub * STRIPE
    for k in range(STRIPE // ZR):
        pltpu.sync_copy(rows_v, agg_sh.at[pl.ds(base + k * ZR, ZR)])
    rem = STRIPE % ZR
    if rem:
        pltpu.sync_copy(
            rows_v.at[pl.ds(0, rem)],
            agg_sh.at[pl.ds(base + (STRIPE // ZR) * ZR, rem)],
        )
    plsc.subcore_barrier()

    bufs = (rows_v, rows_w, rows_x, rows_y)
    HC = CHUNK // 2

    def _edges(chunk_base, size, cpt):
        pltpu.sync_copy(
            src_hbm.at[pl.ds(2 * chunk_base, 2 * size)], src_v.at[pl.ds(0, 2 * size)]
        )
        pltpu.sync_copy(
            dst_hbm.at[pl.ds(2 * chunk_base, 2 * size)], dst_v.at[pl.ds(0, 2 * size)]
        )

        def _issue(hc, b):
            pltpu.async_copy(h_hbm.at[src_v.at[hc]], bufs[b], sems[b])

        def _wait(hc, b):
            pltpu.make_async_copy(h_hbm.at[src_v.at[hc]], bufs[b], sems[b]).wait()

        def _scat(hc, b):
            pltpu.sync_copy(bufs[b], agg_sh.at[dst_v.at[hc]], add=True)

        nh = 2 * cpt
        nt = nh // 4
        for b in range(4):
            _issue(b, b)

        def _qbody(g, carry):
            for b in range(4):
                hc = 4 * g + b
                _wait(hc, b)
                _issue(hc + 4, b)
                _scat(hc, b)
            return carry

        lax.fori_loop(0, nt - 1, _qbody, 0)
        for k in range(4 * (nt - 1), nh):
            b = k % 4
            _wait(k, b)
            if k + 4 < nh:
                _issue(k + 4, b)
            _scat(k, b)

    @pl.when(core == 0)
    def _():
        for w in range(3):
            _edges(sub * S0 + 20 * w, 20, 20)
        _edges(sub * S0 + 60, 20, CPT0 - 60)

    @pl.when(core == 1)
    def _():
        for w in range(3):
            _edges(NS * S0 + sub * S1 + 20 * w, 20, 20)
        _edges(NS * S0 + sub * S1 + 60, 20, CPT1 - 60)

    plsc.subcore_barrier()

    @pl.when(core == 0)
    def _():
        pltpu.sync_copy(agg_sh.at[pl.ds(base, STRIPE)], out0.at[pl.ds(base, STRIPE)])

    @pl.when(core == 1)
    def _():
        pltpu.sync_copy(agg_sh.at[pl.ds(base, STRIPE)], out1.at[pl.ds(base, STRIPE)])


_sc_pass = pl.kernel(
    _sc_body,
    out_type=(
        jax.ShapeDtypeStruct((AGG_ROWS, D), jnp.float32),
        jax.ShapeDtypeStruct((AGG_ROWS, D), jnp.float32),
    ),
    mesh=plsc.VectorSubcoreMesh(
        core_axis_name="c", subcore_axis_name="s", num_cores=NC, num_subcores=NS
    ),
    scratch_types=[
        pltpu.VMEM_SHARED((AGG_ROWS, D), jnp.float32),
        pltpu.VMEM((40, CHUNK // 2), jnp.int32),
        pltpu.VMEM((40, CHUNK // 2), jnp.int32),
        pltpu.VMEM((CHUNK // 2, D), jnp.float32),
        pltpu.VMEM((CHUNK // 2, D), jnp.float32),
        pltpu.VMEM((CHUNK // 2, D), jnp.float32),
        pltpu.VMEM((CHUNK // 2, D), jnp.float32),
    ] + [pltpu.SemaphoreType.DMA] * 4,
)


def _layer_body(a0, a1, h, wrel, wroot, brel, out):
    agg = a0[...] + a1[...]
    out[...] = jnp.maximum(
        jnp.dot(agg, wrel[...], preferred_element_type=jnp.float32)
        + jnp.dot(h[...], wroot[...], preferred_element_type=jnp.float32)
        + brel[...],
        0.0,
    )


_BLK = 1000


def _tc_layer(a0, a1, h, wrel, wroot, brel):
    return pl.pallas_call(
        _layer_body,
        grid=(N_NODES // _BLK,),
        in_specs=[
            pl.BlockSpec((_BLK, D), lambda i: (i, 0)),
            pl.BlockSpec((_BLK, D), lambda i: (i, 0)),
            pl.BlockSpec((_BLK, D), lambda i: (i, 0)),
            pl.BlockSpec((D, D), lambda i: (0, 0)),
            pl.BlockSpec((D, D), lambda i: (0, 0)),
            pl.BlockSpec((1, D), lambda i: (0, 0)),
        ],
        out_specs=pl.BlockSpec((_BLK, D), lambda i: (i, 0)),
        out_shape=jax.ShapeDtypeStruct((N_NODES, D), jnp.float32),
    )(a0, a1, h, wrel, wroot, brel)


def _head_body(h, batch2, snn, w1, b1, w2, b2, linw, linb, fw1, fw2, fb, out):
    memb = (
        batch2[...] == lax.broadcasted_iota(jnp.int32, (N_NODES, N_GRAPHS), 1)
    ).astype(jnp.float32)
    sums = lax.dot_general(
        memb, h[...], (((0,), (0,)), ((), ())),
        preferred_element_type=jnp.float32,
    )
    counts = jnp.sum(memb, axis=0)
    pooled = sums / jnp.maximum(counts, 1.0)[:, None]
    gnn = (
        jnp.dot(pooled, linw[...], preferred_element_type=jnp.float32)
        + linb[...]
    )
    hh = jnp.maximum(
        jnp.dot(snn[...], w1[...], preferred_element_type=jnp.float32)
        + b1[...],
        0.0,
    )
    snl = BETA * (
        jnp.dot(hh, w2[...], preferred_element_type=jnp.float32)
        + b2[...]
    )
    out[...] = (
        jnp.dot(snl, fw1[...], preferred_element_type=jnp.float32)
        + jnp.dot(gnn, fw2[...], preferred_element_type=jnp.float32)
        + fb[...]
    )


def _tc_head(h, batch2, snn, w1, b1, w2, b2, linw, linb, fw1, fw2, fb):
    return pl.pallas_call(
        _head_body,
        out_shape=jax.ShapeDtypeStruct((N_GRAPHS, N_GRAPHS), jnp.float32),
    )(h, batch2, snn, w1, b1, w2, b2, linw, linb, fw1, fw2, fb)


def kernel(snn_batch, x, edge_index, batch, params):
    src = edge_index[0]
    dst = edge_index[1]
    pad = E_PAD - E
    sink_rows = SINK + (jnp.arange(pad, dtype=jnp.int32) % (AGG_ROWS - N_NODES))
    srcp = jnp.concatenate([src, jnp.zeros((pad,), jnp.int32)]).reshape(-1, CHUNK)
    dstp = jnp.concatenate([dst, sink_rows]).reshape(-1, CHUNK)

    def _regions(a):
        c0 = a[: NS * CPT0].reshape(NS, CPT0, CHUNK)
        c1 = a[NS * CPT0 :].reshape(NS, CPT1, CHUNK)
        c0 = jnp.pad(c0, ((0, 0), (0, S0 - CPT0), (0, 0)))
        c1 = jnp.pad(c1, ((0, 0), (0, S1 - CPT1), (0, 0)))
        return jnp.concatenate(
            [c0.reshape(NS * S0, CHUNK), c1.reshape(NS * S1, CHUNK)]
        )

    src_p = _regions(srcp).reshape(-1, CHUNK // 2)
    dst_p = _regions(dstp).reshape(-1, CHUNK // 2)

    h = x
    for i in range(N_LAYERS):
        a0, a1 = _sc_pass(h, src_p, dst_p)
        h = _tc_layer(
            a0,
            a1,
            h,
            params["gnn_Wrel"][i],
            params["gnn_Wroot"][i],
            params["gnn_brel"][i].reshape(1, D),
        )

    return _tc_head(
        h,
        batch.reshape(N_NODES, 1),
        snn_batch,
        params["snn_W1"],
        params["snn_b1"].reshape(1, -1),
        params["snn_W2"],
        params["snn_b2"].reshape(1, -1),
        params["gnn_lin_W"],
        params["gnn_lin_b"].reshape(1, -1),
        params["fusion_W"][:N_GRAPHS],
        params["fusion_W"][N_GRAPHS:],
        params["fusion_b"].reshape(1, -1),
    )


# speedup vs baseline: 1.0208x; 1.0208x over previous
import functools

import jax
import jax.numpy as jnp
from jax import lax
from jax.experimental import pallas as pl
from jax.experimental.pallas import tpu as pltpu
from jax.experimental.pallas import tpu_sc as plsc

N_NODES = 10000
D = 128
E = 320000
N_GRAPHS = 128
N_LAYERS = 7
BETA = 0.85

NC, NS, L = 2, 16, 16
NW = NC * NS
CHUNK = 128
NBUF = 1
CPT0 = 79
CPT1 = 79
S0 = 80
S1 = 80
E_PAD = NS * (CPT0 + CPT1) * CHUNK
AGG_ROWS = 10112
STRIPE = AGG_ROWS // NS
SINK = N_NODES

def _sc_body(h_hbm, src_hbm, dst_hbm, out0, out1, agg_sh, src_v, dst_v, rows_v,
             rows_w, rows_x, *sems):
    core = lax.axis_index("c")
    sub = lax.axis_index("s")
    wid = core * NS + sub

    def _zrow(i, carry):
        for c in range(D // L):
            rows_v[i, pl.ds(c * L, L)] = jnp.zeros((L,), jnp.float32)
        return carry

    ZR = CHUNK // 2
    lax.fori_loop(0, ZR, _zrow, 0)
    base = sub * STRIPE
    for k in range(STRIPE // ZR):
        pltpu.sync_copy(rows_v, agg_sh.at[pl.ds(base + k * ZR, ZR)])
    rem = STRIPE % ZR
    if rem:
        pltpu.sync_copy(
            rows_v.at[pl.ds(0, rem)],
            agg_sh.at[pl.ds(base + (STRIPE // ZR) * ZR, rem)],
        )
    plsc.subcore_barrier()

    bufs = (rows_v, rows_w, rows_x)
    HC = CHUNK // 2

    def _edges(chunk_base, size, cpt):
        pltpu.sync_copy(
            src_hbm.at[pl.ds(2 * chunk_base, 2 * size)], src_v.at[pl.ds(0, 2 * size)]
        )
        pltpu.sync_copy(
            dst_hbm.at[pl.ds(2 * chunk_base, 2 * size)], dst_v.at[pl.ds(0, 2 * size)]
        )

        def _issue(hc, b):
            pltpu.async_copy(h_hbm.at[src_v.at[hc]], bufs[b], sems[b])

        def _wait(hc, b):
            pltpu.make_async_copy(h_hbm.at[src_v.at[hc]], bufs[b], sems[b]).wait()

        def _scat(hc, b):
            pltpu.sync_copy(bufs[b], agg_sh.at[dst_v.at[hc]], add=True)

        nh = 2 * cpt
        nt = nh // 3
        for b in range(3):
            _issue(b, b)

        def _qbody(g, carry):
            for b in range(3):
                hc = 3 * g + b
                _wait(hc, b)
                _issue(hc + 3, b)
                _scat(hc, b)
            return carry

        lax.fori_loop(0, nt - 1, _qbody, 0)
        for k in range(3 * (nt - 1), nh):
            b = k % 3
            _wait(k, b)
            if k + 3 < nh:
                _issue(k + 3, b)
            _scat(k, b)

    @pl.when(core == 0)
    def _():
        _edges(sub * S0, 40, 40)
        _edges(sub * S0 + 40, 40, CPT0 - 40)

    @pl.when(core == 1)
    def _():
        _edges(NS * S0 + sub * S1, 40, 40)
        _edges(NS * S0 + sub * S1 + 40, 40, CPT1 - 40)

    plsc.subcore_barrier()

    @pl.when(core == 0)
    def _():
        pltpu.sync_copy(agg_sh.at[pl.ds(base, STRIPE)], out0.at[pl.ds(base, STRIPE)])

    @pl.when(core == 1)
    def _():
        pltpu.sync_copy(agg_sh.at[pl.ds(base, STRIPE)], out1.at[pl.ds(base, STRIPE)])


_sc_pass = pl.kernel(
    _sc_body,
    out_type=(
        jax.ShapeDtypeStruct((AGG_ROWS, D), jnp.float32),
        jax.ShapeDtypeStruct((AGG_ROWS, D), jnp.float32),
    ),
    mesh=plsc.VectorSubcoreMesh(
        core_axis_name="c", subcore_axis_name="s", num_cores=NC, num_subcores=NS
    ),
    scratch_types=[
        pltpu.VMEM_SHARED((AGG_ROWS, D), jnp.float32),
        pltpu.VMEM((80, CHUNK // 2), jnp.int32),
        pltpu.VMEM((80, CHUNK // 2), jnp.int32),
        pltpu.VMEM((CHUNK // 2, D), jnp.float32),
        pltpu.VMEM((CHUNK // 2, D), jnp.float32),
        pltpu.VMEM((CHUNK // 2, D), jnp.float32),
    ] + [pltpu.SemaphoreType.DMA] * 3,
)


def _layer_body(a0, a1, h, wrel, wroot, brel, out):
    agg = a0[...] + a1[...]
    out[...] = jnp.maximum(
        jnp.dot(agg, wrel[...], preferred_element_type=jnp.float32)
        + jnp.dot(h[...], wroot[...], preferred_element_type=jnp.float32)
        + brel[...],
        0.0,
    )


_BLK = 1000


def _tc_layer(a0, a1, h, wrel, wroot, brel):
    return pl.pallas_call(
        _layer_body,
        grid=(N_NODES // _BLK,),
        in_specs=[
            pl.BlockSpec((_BLK, D), lambda i: (i, 0)),
            pl.BlockSpec((_BLK, D), lambda i: (i, 0)),
            pl.BlockSpec((_BLK, D), lambda i: (i, 0)),
            pl.BlockSpec((D, D), lambda i: (0, 0)),
            pl.BlockSpec((D, D), lambda i: (0, 0)),
            pl.BlockSpec((1, D), lambda i: (0, 0)),
        ],
        out_specs=pl.BlockSpec((_BLK, D), lambda i: (i, 0)),
        out_shape=jax.ShapeDtypeStruct((N_NODES, D), jnp.float32),
    )(a0, a1, h, wrel, wroot, brel)


def _head_body(h, batch2, snn, w1, b1, w2, b2, linw, linb, fw1, fw2, fb, out):
    memb = (
        batch2[...] == lax.broadcasted_iota(jnp.int32, (N_NODES, N_GRAPHS), 1)
    ).astype(jnp.float32)
    sums = lax.dot_general(
        memb, h[...], (((0,), (0,)), ((), ())),
        preferred_element_type=jnp.float32,
    )
    counts = jnp.sum(memb, axis=0)
    pooled = sums / jnp.maximum(counts, 1.0)[:, None]
    gnn = (
        jnp.dot(pooled, linw[...], preferred_element_type=jnp.float32)
        + linb[...]
    )
    hh = jnp.maximum(
        jnp.dot(snn[...], w1[...], preferred_element_type=jnp.float32)
        + b1[...],
        0.0,
    )
    snl = BETA * (
        jnp.dot(hh, w2[...], preferred_element_type=jnp.float32)
        + b2[...]
    )
    out[...] = (
        jnp.dot(snl, fw1[...], preferred_element_type=jnp.float32)
        + jnp.dot(gnn, fw2[...], preferred_element_type=jnp.float32)
        + fb[...]
    )


def _tc_head(h, batch2, snn, w1, b1, w2, b2, linw, linb, fw1, fw2, fb):
    return pl.pallas_call(
        _head_body,
        out_shape=jax.ShapeDtypeStruct((N_GRAPHS, N_GRAPHS), jnp.float32),
    )(h, batch2, snn, w1, b1, w2, b2, linw, linb, fw1, fw2, fb)


def kernel(snn_batch, x, edge_index, batch, params):
    src = edge_index[0]
    dst = edge_index[1]
    pad = E_PAD - E
    sink_rows = SINK + (jnp.arange(pad, dtype=jnp.int32) % (AGG_ROWS - N_NODES))
    srcp = jnp.concatenate([src, jnp.zeros((pad,), jnp.int32)]).reshape(-1, CHUNK)
    dstp = jnp.concatenate([dst, sink_rows]).reshape(-1, CHUNK)

    def _regions(a):
        c0 = a[: NS * CPT0].reshape(NS, CPT0, CHUNK)
        c1 = a[NS * CPT0 :].reshape(NS, CPT1, CHUNK)
        c0 = jnp.pad(c0, ((0, 0), (0, S0 - CPT0), (0, 0)))
        c1 = jnp.pad(c1, ((0, 0), (0, S1 - CPT1), (0, 0)))
        return jnp.concatenate(
            [c0.reshape(NS * S0, CHUNK), c1.reshape(NS * S1, CHUNK)]
        )

    src_p = _regions(srcp).reshape(-1, CHUNK // 2)
    dst_p = _regions(dstp).reshape(-1, CHUNK // 2)

    h = x
    for i in range(N_LAYERS):
        a0, a1 = _sc_pass(h, src_p, dst_p)
        h = _tc_layer(
            a0,
            a1,
            h,
            params["gnn_Wrel"][i],
            params["gnn_Wroot"][i],
            params["gnn_brel"][i].reshape(1, D),
        )

    return _tc_head(
        h,
        batch.reshape(N_NODES, 1),
        snn_batch,
        params["snn_W1"],
        params["snn_b1"].reshape(1, -1),
        params["snn_W2"],
        params["snn_b2"].reshape(1, -1),
        params["gnn_lin_W"],
        params["gnn_lin_b"].reshape(1, -1),
        params["fusion_W"][:N_GRAPHS],
        params["fusion_W"][N_GRAPHS:],
        params["fusion_b"].reshape(1, -1),
    )
